# baseline (speedup 1.0000x reference)
import functools
import numpy as np
import jax
import jax.numpy as jnp
from jax import lax
from jax.experimental import pallas as pl
from jax.experimental.pallas import tpu as pltpu

H = 256
F = 128
N = 10000
S = 1024
K = 32
R2 = np.float32(0.1 ** 2)
ROWB = 2000
NBLK = N // ROWB
NPAD = 10240
CB = NPAD // NBLK
SB = 128
NSB = S // SB
NEG_INF = np.float32(-np.inf)


def _embed_consts():
    eff = 84
    nfreq = eff // 2
    f = np.arange(H)
    d = np.minimum(f // (2 * nfreq), 2)
    r = f % (2 * nfreq)
    is_cos = (r >= nfreq).astype(np.float32)
    kk = r % nfreq
    omega = (1.0 / (10000.0 ** (np.arange(0, eff, 2).astype(np.float32) / np.float32(eff)))).astype(np.float32)
    omega_map = omega[kk]
    valid = (f < 6 * nfreq).astype(np.float32)
    return (omega_map.reshape(1, H), is_cos.reshape(1, H), valid.reshape(1, H),
            d.astype(np.int32).reshape(1, H))


_OMEGA, _COSM, _VALIDM, _DIMM = _embed_consts()


def _gelu_exact(v):
    z = v * np.float32(0.7071067811865476)
    a = jnp.abs(z)
    t = np.float32(1.0) / (np.float32(1.0) + np.float32(0.3275911) * a)
    p = np.float32(1.061405429)
    p = p * t + np.float32(-1.453152027)
    p = p * t + np.float32(1.421413741)
    p = p * t + np.float32(-0.284496736)
    p = p * t + np.float32(0.254829592)
    erf_a = np.float32(1.0) - (p * t) * jnp.exp(-a * a)
    erf = jnp.sign(z) * erf_a
    return np.float32(0.5) * v * (np.float32(1.0) + erf)


def _dense_a_body(pos_ref, feat_ref, sup5_ref, pos5_ref, w_in_ref, b_in_ref,
                  w_msg1_ref, omega_ref, cosm_ref, validm_ref, dimm_ref,
                  x_ref, y_ref, d2_ref):
    pos = pos_ref[...]
    px = pos[:, 0:1]
    py = pos[:, 1:2]
    pz = pos[:, 2:3]
    dimm = dimm_ref[...]
    posd = jnp.where(dimm == 0, px, jnp.where(dimm == 1, py, pz))
    arg = posd * omega_ref[...]
    emb = jnp.where(cosm_ref[...] > 0, jnp.cos(arg), jnp.sin(arg)) * validm_ref[...]
    x = emb + lax.dot_general(feat_ref[...], w_in_ref[...],
                              (((1,), (1,)), ((), ())),
                              preferred_element_type=jnp.float32) + b_in_ref[...]
    w1s = w_msg1_ref[:, :H]
    y = lax.dot_general(x, w1s, (((1,), (1,)), ((), ())),
                        preferred_element_type=jnp.float32)
    x_ref[...] = x
    y_ref[...] = y
    d2_ref[...] = lax.dot_general(sup5_ref[...], pos5_ref[...],
                                  (((1,), (1,)), ((), ())),
                                  preferred_element_type=jnp.float32)


def _dense_c_body(x_ref, gath_ref, w_ref, w_msg1_ref, b_msg1_ref,
                  w_msg2_ref, b_msg2_ref, w_proj_ref, b_proj_ref, out_ref):
    xs = x_ref[...]
    w1d = w_msg1_ref[:, H:]
    dstc = lax.dot_general(xs, w1d, (((1,), (1,)), ((), ())),
                           preferred_element_type=jnp.float32) + b_msg1_ref[...]
    drep = jnp.reshape(jnp.broadcast_to(dstc[:, None, :], (SB, K, H)), (SB * K, H))
    g = _gelu_exact(gath_ref[...] + drep) * w_ref[...]
    hs = jnp.sum(jnp.reshape(g, (SB, K, H)), axis=1)
    agg = lax.dot_general(hs, w_msg2_ref[...], (((1,), (1,)), ((), ())),
                          preferred_element_type=jnp.float32) + b_msg2_ref[...]
    wpa = w_proj_ref[:, :H]
    wps = w_proj_ref[:, H:]
    out = (lax.dot_general(agg, wpa, (((1,), (1,)), ((), ())),
                           preferred_element_type=jnp.float32)
           + lax.dot_general(xs, wps, (((1,), (1,)), ((), ())),
                             preferred_element_type=jnp.float32)
           + b_proj_ref[...])
    out_ref[0] = out


def _full(shape):
    n = len(shape)
    return pl.BlockSpec(shape, lambda i: (0,) * n)


def kernel(input_pos, input_feat, supernode_idx, W_in, b_in, W_msg1, b_msg1,
           W_msg2, b_msg2, W_proj, b_proj):
    pos = input_pos
    p2 = jnp.sum(pos * pos, axis=1, keepdims=True)
    sup_pos = pos[:S]
    sup2 = p2[:S]
    sup5 = jnp.concatenate([np.float32(-2.0) * sup_pos, sup2,
                            jnp.ones((S, 1), jnp.float32)], axis=1)
    pos5 = jnp.concatenate([pos, jnp.ones((N, 1), jnp.float32), p2], axis=1)
    pad5 = jnp.tile(jnp.asarray([[0.0, 0.0, 0.0, 1.0, 1e9]], jnp.float32),
                    (NPAD - N, 1))
    pos5 = jnp.concatenate([pos5, pad5], axis=0)
    b_in2 = b_in.reshape(1, H)

    x, y, d2 = pl.pallas_call(
        _dense_a_body,
        grid=(NBLK,),
        in_specs=[
            pl.BlockSpec((ROWB, 3), lambda i: (i, 0)),
            pl.BlockSpec((ROWB, F), lambda i: (i, 0)),
            _full((S, 5)),
            pl.BlockSpec((CB, 5), lambda i: (i, 0)),
            _full((H, F)),
            _full((1, H)),
            _full((H, 2 * H)),
            _full((1, H)),
            _full((1, H)),
            _full((1, H)),
            _full((1, H)),
        ],
        out_specs=[
            pl.BlockSpec((ROWB, H), lambda i: (i, 0)),
            pl.BlockSpec((ROWB, H), lambda i: (i, 0)),
            pl.BlockSpec((S, CB), lambda i: (0, i)),
        ],
        out_shape=[
            jax.ShapeDtypeStruct((N, H), jnp.float32),
            jax.ShapeDtypeStruct((N, H), jnp.float32),
            jax.ShapeDtypeStruct((S, NPAD), jnp.float32),
        ],
    )(pos, input_feat, sup5, pos5, W_in, b_in2, W_msg1,
      jnp.asarray(_OMEGA), jnp.asarray(_COSM), jnp.asarray(_VALIDM), jnp.asarray(_DIMM))

    d2 = d2[:, :N]
    score = jnp.where(d2 <= R2, -d2, NEG_INF)
    top_vals, nbr_idx = lax.top_k(score, K)
    valid = top_vals > NEG_INF
    nbr_idx = jnp.where(valid, nbr_idx, 0)
    counts = jnp.maximum(valid.sum(axis=-1).astype(jnp.float32), 1.0)
    w = valid.astype(jnp.float32) / counts[:, None]
    gath = y[nbr_idx.reshape(-1)]

    out = pl.pallas_call(
        _dense_c_body,
        grid=(NSB,),
        in_specs=[
            pl.BlockSpec((SB, H), lambda i: (i, 0)),
            pl.BlockSpec((SB * K, H), lambda i: (i, 0)),
            pl.BlockSpec((SB * K, 1), lambda i: (i, 0)),
            _full((H, 2 * H)),
            _full((1, H)),
            _full((H, H)),
            _full((1, H)),
            _full((H, 2 * H)),
            _full((1, H)),
        ],
        out_specs=pl.BlockSpec((1, SB, H), lambda i: (0, i, 0)),
        out_shape=jax.ShapeDtypeStruct((1, S, H), jnp.float32),
    )(x[:S], gath, w.reshape(S * K, 1), W_msg1, b_msg1.reshape(1, H),
      W_msg2, b_msg2.reshape(1, H), W_proj, b_proj.reshape(1, H))
    return out


# speedup vs baseline: 2.6679x; 2.6679x over previous
import functools
import numpy as np
import jax
import jax.numpy as jnp
from jax import lax
from jax.experimental import pallas as pl
from jax.experimental.pallas import tpu as pltpu
from jax.experimental.pallas import tpu_sc as plsc

H = 256
F = 128
N = 10000
S = 1024
K = 32
R2 = np.float32(0.1 ** 2)
ROWB = 2000
NBLK = N // ROWB
NPAD = 10240
CB = NPAD // NBLK
SB = 128
NSB = S // SB
NEG_INF = np.float32(-np.inf)


def _embed_consts():
    eff = 84
    nfreq = eff // 2
    f = np.arange(H)
    d = np.minimum(f // (2 * nfreq), 2)
    r = f % (2 * nfreq)
    is_cos = (r >= nfreq).astype(np.float32)
    kk = r % nfreq
    omega = (1.0 / (10000.0 ** (np.arange(0, eff, 2).astype(np.float32) / np.float32(eff)))).astype(np.float32)
    omega_map = omega[kk]
    valid = (f < 6 * nfreq).astype(np.float32)
    return (omega_map.reshape(1, H), is_cos.reshape(1, H), valid.reshape(1, H),
            d.astype(np.int32).reshape(1, H))


_OMEGA, _COSM, _VALIDM, _DIMM = _embed_consts()


NW = 32
RPW = S // NW
CAP = 256
L = 16
NVS = NPAD // L
EPB = S * K // NW
CH = 128
NCH = EPB // CH


def _lane(v, j=0):
    return lax.squeeze(lax.slice(v, (j,), (j + 1,)), (0,))


def _splat(s, dtype=jnp.int32):
    return jnp.full((L,), s, dtype)


def _sel_body(px_hbm, py_hbm, pz_hbm, nbr_hbm, w_hbm,
              px, py, pz, cand_d, cand_i, nbr_v, w_v):
    wid = lax.axis_index("s") * 2 + lax.axis_index("c")
    pltpu.sync_copy(px_hbm, px)
    pltpu.sync_copy(py_hbm, py)
    pltpu.sync_copy(pz_hbm, pz)
    iota = lax.iota(jnp.int32, L)
    zero_i = jnp.zeros((L,), jnp.int32)
    rowbase = wid * RPW

    cap_sp = jnp.full((L,), CAP, jnp.int32)
    capm1_sp = jnp.full((L,), CAP - 1, jnp.int32)

    def row_body(r, carry_dummy):
        i = rowbase + r
        sx = _splat(px[pl.ds(i, L)][0], jnp.float32)
        sy = _splat(py[pl.ds(i, L)][0], jnp.float32)
        sz = _splat(pz[pl.ds(i, L)][0], jnp.float32)

        def scan_body(j, cnt_sp):
            off = j * L
            dx = px[pl.ds(off, L)] - sx
            dy = py[pl.ds(off, L)] - sy
            dz = pz[pl.ds(off, L)] - sz
            d2 = (dx * dx + dy * dy) + dz * dz
            m = d2 <= R2
            incl = plsc.cumsum(m.astype(jnp.int32))
            c = _lane(incl, L - 1)

            @pl.when(c > 0)
            def _():
                tgt = jnp.minimum(cnt_sp + incl - 1, capm1_sp)
                plsc.store_scatter(cand_d, [tgt], d2, mask=m)
                plsc.store_scatter(cand_i, [tgt], iota + off, mask=m)

            return jnp.minimum(cnt_sp + _splat(c), cap_sp)

        cnt_sp = lax.fori_loop(0, NVS, scan_body, zero_i)
        cnt = _lane(cnt_sp)
        nvc = (cnt + L - 1) // L

        def count_lt(thr_sp):
            def cbody(kk, acc):
                uv = plsc.bitcast(cand_d[pl.ds(kk * L, L)], jnp.int32)
                valid = (iota + kk * L) < cnt_sp
                mlt = (uv < thr_sp) & valid
                return acc + plsc.cumsum(mlt.astype(jnp.int32))
            acc = lax.fori_loop(0, nvc, cbody, zero_i)
            return _splat(_lane(acc, L - 1))

        def bit_body(bi, vthr):
            tryv = vthr | _splat(jnp.int32(1) << (30 - bi))
            return jnp.where(count_lt(tryv) < K, tryv, vthr)

        vthr = lax.fori_loop(0, 31, bit_body, zero_i)
        c_lt = count_lt(vthr)
        kp_sp = _splat(K, jnp.int32) - c_lt

        nbr_v[pl.ds(r * K, L)] = zero_i
        nbr_v[pl.ds(r * K + L, L)] = zero_i

        rk_sp = _splat(r * K)

        def emit_body(kk, carry):
            outp_sp, eqc = carry
            uv = plsc.bitcast(cand_d[pl.ds(kk * L, L)], jnp.int32)
            valid = (iota + kk * L) < cnt_sp
            lt = (uv < vthr) & valid
            eq = (uv == vthr) & valid
            pe = plsc.cumsum(eq.astype(jnp.int32)) + eqc
            sel = lt | (eq & (pe <= kp_sp))
            selc = plsc.cumsum(sel.astype(jnp.int32))
            tgt = rk_sp + outp_sp + selc - 1
            plsc.store_scatter(nbr_v, [tgt], cand_i[pl.ds(kk * L, L)], mask=sel)
            outp_sp = outp_sp + _splat(_lane(selc, L - 1))
            eqc = _splat(_lane(pe, L - 1))
            return outp_sp, eqc

        lax.fori_loop(0, nvc, emit_body, (zero_i, zero_i))

        ksel = jnp.minimum(cnt, K)
        inv_sp = jnp.full((L,), 1.0, jnp.float32) / _splat(ksel.astype(jnp.float32), jnp.float32)
        ksp = _splat(ksel)
        zf = jnp.zeros((L,), jnp.float32)
        w_v[pl.ds(r * K, L)] = jnp.where(iota < ksp, inv_sp, zf)
        w_v[pl.ds(r * K + L, L)] = jnp.where(iota + L < ksp, inv_sp, zf)
        return 0

    lax.fori_loop(0, RPW, row_body, 0)
    pltpu.sync_copy(nbr_v, nbr_hbm.at[pl.ds(wid * EPB, EPB)])
    pltpu.sync_copy(w_v, w_hbm.at[pl.ds(wid * EPB, EPB)])


def _gather_body(y_hbm, nbr_hbm, out_hbm, idx_v, gbuf, semg, sems):
    wid = lax.axis_index("s") * 2 + lax.axis_index("c")
    base = wid * EPB
    pltpu.sync_copy(nbr_hbm.at[pl.ds(base, EPB)], idx_v)

    def gcopy(b):
        return pltpu.make_async_copy(
            y_hbm.at[idx_v.at[pl.ds(b * CH, CH)]], gbuf.at[b % 2], semg.at[b % 2])

    def scopy(b):
        return pltpu.make_async_copy(
            gbuf.at[b % 2], out_hbm.at[pl.ds(base + b * CH, CH)], sems.at[b % 2])

    gcopy(0).start()
    for b in range(NCH):
        gcopy(b).wait()
        if b + 1 < NCH:
            if b >= 1:
                scopy(b - 1).wait()
            gcopy(b + 1).start()
        scopy(b).start()
    scopy(NCH - 2).wait()
    scopy(NCH - 1).wait()


def _gelu_exact(v):
    z = v * np.float32(0.7071067811865476)
    a = jnp.abs(z)
    t = np.float32(1.0) / (np.float32(1.0) + np.float32(0.3275911) * a)
    p = np.float32(1.061405429)
    p = p * t + np.float32(-1.453152027)
    p = p * t + np.float32(1.421413741)
    p = p * t + np.float32(-0.284496736)
    p = p * t + np.float32(0.254829592)
    erf_a = np.float32(1.0) - (p * t) * jnp.exp(-a * a)
    erf = jnp.sign(z) * erf_a
    return np.float32(0.5) * v * (np.float32(1.0) + erf)


def _dense_a_body(pos_ref, feat_ref, w_in_ref, b_in_ref,
                  w_msg1_ref, omega_ref, cosm_ref, validm_ref, dimm_ref,
                  x_ref, y_ref):
    pos = pos_ref[...]
    px = pos[:, 0:1]
    py = pos[:, 1:2]
    pz = pos[:, 2:3]
    dimm = dimm_ref[...]
    posd = jnp.where(dimm == 0, px, jnp.where(dimm == 1, py, pz))
    arg = posd * omega_ref[...]
    emb = jnp.where(cosm_ref[...] > 0, jnp.cos(arg), jnp.sin(arg)) * validm_ref[...]
    x = emb + lax.dot_general(feat_ref[...], w_in_ref[...],
                              (((1,), (1,)), ((), ())),
                              preferred_element_type=jnp.float32) + b_in_ref[...]
    w1s = w_msg1_ref[:, :H]
    y = lax.dot_general(x, w1s, (((1,), (1,)), ((), ())),
                        preferred_element_type=jnp.float32)
    x_ref[...] = x
    y_ref[...] = y


def _dense_c_body(x_ref, gath_ref, w_ref, w_msg1_ref, b_msg1_ref,
                  w_msg2_ref, b_msg2_ref, w_proj_ref, b_proj_ref, out_ref):
    xs = x_ref[...]
    w1d = w_msg1_ref[:, H:]
    dstc = lax.dot_general(xs, w1d, (((1,), (1,)), ((), ())),
                           preferred_element_type=jnp.float32) + b_msg1_ref[...]
    drep = jnp.reshape(jnp.broadcast_to(dstc[:, None, :], (SB, K, H)), (SB * K, H))
    g = _gelu_exact(gath_ref[...] + drep) * w_ref[...]
    hs = jnp.sum(jnp.reshape(g, (SB, K, H)), axis=1)
    agg = lax.dot_general(hs, w_msg2_ref[...], (((1,), (1,)), ((), ())),
                          preferred_element_type=jnp.float32) + b_msg2_ref[...]
    wpa = w_proj_ref[:, :H]
    wps = w_proj_ref[:, H:]
    out = (lax.dot_general(agg, wpa, (((1,), (1,)), ((), ())),
                           preferred_element_type=jnp.float32)
           + lax.dot_general(xs, wps, (((1,), (1,)), ((), ())),
                             preferred_element_type=jnp.float32)
           + b_proj_ref[...])
    out_ref[0] = out


def _full(shape):
    n = len(shape)
    return pl.BlockSpec(shape, lambda i: (0,) * n)


def kernel(input_pos, input_feat, supernode_idx, W_in, b_in, W_msg1, b_msg1,
           W_msg2, b_msg2, W_proj, b_proj):
    pos = input_pos
    b_in2 = b_in.reshape(1, H)
    pos_pad = jnp.concatenate(
        [pos, jnp.full((NPAD - N, 3), 100.0, jnp.float32)], axis=0)
    px_a = pos_pad[:, 0]
    py_a = pos_pad[:, 1]
    pz_a = pos_pad[:, 2]

    nbr, w = pl.kernel(
        _sel_body,
        out_type=[jax.ShapeDtypeStruct((S * K,), jnp.int32),
                  jax.ShapeDtypeStruct((S * K,), jnp.float32)],
        mesh=plsc.VectorSubcoreMesh(core_axis_name="c", subcore_axis_name="s"),
        compiler_params=pltpu.CompilerParams(needs_layout_passes=False),
        scratch_types=[
            pltpu.VMEM((NPAD,), jnp.float32),
            pltpu.VMEM((NPAD,), jnp.float32),
            pltpu.VMEM((NPAD,), jnp.float32),
            pltpu.VMEM((CAP,), jnp.float32),
            pltpu.VMEM((CAP,), jnp.int32),
            pltpu.VMEM((EPB,), jnp.int32),
            pltpu.VMEM((EPB,), jnp.float32),
        ],
    )(px_a, py_a, pz_a)

    x, y = pl.pallas_call(
        _dense_a_body,
        grid=(NBLK,),
        in_specs=[
            pl.BlockSpec((ROWB, 3), lambda i: (i, 0)),
            pl.BlockSpec((ROWB, F), lambda i: (i, 0)),
            _full((H, F)),
            _full((1, H)),
            _full((H, 2 * H)),
            _full((1, H)),
            _full((1, H)),
            _full((1, H)),
            _full((1, H)),
        ],
        out_specs=[
            pl.BlockSpec((ROWB, H), lambda i: (i, 0)),
            pl.BlockSpec((ROWB, H), lambda i: (i, 0)),
        ],
        out_shape=[
            jax.ShapeDtypeStruct((N, H), jnp.float32),
            jax.ShapeDtypeStruct((N, H), jnp.float32),
        ],
    )(pos, input_feat, W_in, b_in2, W_msg1,
      jnp.asarray(_OMEGA), jnp.asarray(_COSM), jnp.asarray(_VALIDM), jnp.asarray(_DIMM))

    gath = pl.kernel(
        _gather_body,
        out_type=jax.ShapeDtypeStruct((S * K, H), jnp.float32),
        mesh=plsc.VectorSubcoreMesh(core_axis_name="c", subcore_axis_name="s"),
        compiler_params=pltpu.CompilerParams(needs_layout_passes=False),
        scratch_types=[
            pltpu.VMEM((EPB,), jnp.int32),
            pltpu.VMEM((2, CH, H), jnp.float32),
            pltpu.SemaphoreType.DMA((2,)),
            pltpu.SemaphoreType.DMA((2,)),
        ],
    )(y, nbr)

    out = pl.pallas_call(
        _dense_c_body,
        grid=(NSB,),
        in_specs=[
            pl.BlockSpec((SB, H), lambda i: (i, 0)),
            pl.BlockSpec((SB * K, H), lambda i: (i, 0)),
            pl.BlockSpec((SB * K, 1), lambda i: (i, 0)),
            _full((H, 2 * H)),
            _full((1, H)),
            _full((H, H)),
            _full((1, H)),
            _full((H, 2 * H)),
            _full((1, H)),
        ],
        out_specs=pl.BlockSpec((1, SB, H), lambda i: (0, i, 0)),
        out_shape=jax.ShapeDtypeStruct((1, S, H), jnp.float32),
    )(x[:S], gath, w.reshape(S * K, 1), W_msg1, b_msg1.reshape(1, H),
      W_msg2, b_msg2.reshape(1, H), W_proj, b_proj.reshape(1, H))
    return out


# speedup vs baseline: 5.7410x; 2.1518x over previous
import functools
import numpy as np
import jax
import jax.numpy as jnp
from jax import lax
from jax.experimental import pallas as pl
from jax.experimental.pallas import tpu as pltpu
from jax.experimental.pallas import tpu_sc as plsc

H = 256
F = 128
N = 10000
S = 1024
K = 32
R2 = np.float32(0.1 ** 2)
ROWB = 2000
NBLK = N // ROWB
NPAD = 10240
CB = NPAD // NBLK
SB = 128
NSB = S // SB
NEG_INF = np.float32(-np.inf)


def _embed_consts():
    eff = 84
    nfreq = eff // 2
    f = np.arange(H)
    d = np.minimum(f // (2 * nfreq), 2)
    r = f % (2 * nfreq)
    is_cos = (r >= nfreq).astype(np.float32)
    kk = r % nfreq
    omega = (1.0 / (10000.0 ** (np.arange(0, eff, 2).astype(np.float32) / np.float32(eff)))).astype(np.float32)
    omega_map = omega[kk]
    valid = (f < 6 * nfreq).astype(np.float32)
    return (omega_map.reshape(1, H), is_cos.reshape(1, H), valid.reshape(1, H),
            d.astype(np.int32).reshape(1, H))


_OMEGA, _COSM, _VALIDM, _DIMM = _embed_consts()


NW = 32
RPW = S // NW
CAP = 256
L = 16
GRP = 8
NVS = NPAD // L
EPB = S * K // NW
CH = 128
NCH = EPB // CH


def _lane(v, j=0):
    return lax.squeeze(lax.slice(v, (j,), (j + 1,)), (0,))


def _splat(s, dtype=jnp.int32):
    return jnp.full((L,), s, dtype)


def _sel_body(px_hbm, py_hbm, pz_hbm, nbr_hbm, w_hbm,
              px, py, pz, cand_d, cand_i, nbr_v, w_v):
    wid = lax.axis_index("s") * 2 + lax.axis_index("c")
    pltpu.sync_copy(px_hbm, px)
    pltpu.sync_copy(py_hbm, py)
    pltpu.sync_copy(pz_hbm, pz)
    iota = lax.iota(jnp.int32, L)
    zero_i = jnp.zeros((L,), jnp.int32)
    rowbase = wid * RPW

    cap_sp = jnp.full((L,), CAP, jnp.int32)
    capm1_sp = jnp.full((L,), CAP - 1, jnp.int32)

    def row_body(r, carry_dummy):
        i = rowbase + r
        sx = _splat(px[pl.ds(i, L)][0], jnp.float32)
        sy = _splat(py[pl.ds(i, L)][0], jnp.float32)
        sz = _splat(pz[pl.ds(i, L)][0], jnp.float32)

        def group_body(g, cnt_sp):
            off0 = g * (GRP * L)
            d2s = []
            ms = []
            anym = None
            for u in range(GRP):
                off = off0 + u * L
                dx = px[pl.ds(off, L)] - sx
                dy = py[pl.ds(off, L)] - sy
                dz = pz[pl.ds(off, L)] - sz
                d2 = (dx * dx + dy * dy) + dz * dz
                m = d2 <= R2
                d2s.append(d2)
                ms.append(m)
                anym = m if anym is None else (anym | m)
            anyc = _lane(plsc.cumsum(anym.astype(jnp.int32)), L - 1)

            def slow(cs):
                for u in range(GRP):
                    incl = plsc.cumsum(ms[u].astype(jnp.int32))
                    cu = _lane(incl, L - 1)

                    def hit(cs2, incl=incl, u=u, cu=cu, off=off0 + u * L):
                        tgt = jnp.minimum(cs2 + incl - 1, capm1_sp)
                        plsc.store_scatter(cand_d, [tgt], d2s[u], mask=ms[u])
                        plsc.store_scatter(cand_i, [tgt], iota + off, mask=ms[u])
                        return jnp.minimum(cs2 + _splat(cu), cap_sp)

                    cs = lax.cond(cu > 0, hit, lambda cs2: cs2, cs)
                return cs

            return lax.cond(anyc > 0, slow, lambda cs: cs, cnt_sp)

        cnt_sp = lax.fori_loop(0, NVS // GRP, group_body, zero_i)
        cnt = _lane(cnt_sp)
        nvc = (cnt + L - 1) // L

        def count_lt(thr_sp):
            def cbody(kk, acc):
                uv = plsc.bitcast(cand_d[pl.ds(kk * L, L)], jnp.int32)
                valid = (iota + kk * L) < cnt_sp
                mlt = (uv < thr_sp) & valid
                return acc + plsc.cumsum(mlt.astype(jnp.int32))
            acc = lax.fori_loop(0, nvc, cbody, zero_i)
            return _splat(_lane(acc, L - 1))

        def bit_body(bi, vthr):
            tryv = vthr | _splat(jnp.int32(1) << (30 - bi))
            return jnp.where(count_lt(tryv) < K, tryv, vthr)

        vthr = lax.fori_loop(0, 31, bit_body, zero_i)
        c_lt = count_lt(vthr)
        kp_sp = _splat(K, jnp.int32) - c_lt

        nbr_v[pl.ds(r * K, L)] = zero_i
        nbr_v[pl.ds(r * K + L, L)] = zero_i

        rk_sp = _splat(r * K)

        def emit_body(kk, carry):
            outp_sp, eqc = carry
            uv = plsc.bitcast(cand_d[pl.ds(kk * L, L)], jnp.int32)
            valid = (iota + kk * L) < cnt_sp
            lt = (uv < vthr) & valid
            eq = (uv == vthr) & valid
            pe = plsc.cumsum(eq.astype(jnp.int32)) + eqc
            sel = lt | (eq & (pe <= kp_sp))
            selc = plsc.cumsum(sel.astype(jnp.int32))
            tgt = rk_sp + outp_sp + selc - 1
            plsc.store_scatter(nbr_v, [tgt], cand_i[pl.ds(kk * L, L)], mask=sel)
            outp_sp = outp_sp + _splat(_lane(selc, L - 1))
            eqc = _splat(_lane(pe, L - 1))
            return outp_sp, eqc

        lax.fori_loop(0, nvc, emit_body, (zero_i, zero_i))

        ksel = jnp.minimum(cnt, K)
        inv_sp = jnp.full((L,), 1.0, jnp.float32) / _splat(ksel.astype(jnp.float32), jnp.float32)
        ksp = _splat(ksel)
        zf = jnp.zeros((L,), jnp.float32)
        w_v[pl.ds(r * K, L)] = jnp.where(iota < ksp, inv_sp, zf)
        w_v[pl.ds(r * K + L, L)] = jnp.where(iota + L < ksp, inv_sp, zf)
        return 0

    lax.fori_loop(0, RPW, row_body, 0)
    pltpu.sync_copy(nbr_v, nbr_hbm.at[pl.ds(wid * EPB, EPB)])
    pltpu.sync_copy(w_v, w_hbm.at[pl.ds(wid * EPB, EPB)])


def _gather_body(y_hbm, nbr_hbm, out_hbm, idx_v, gbuf, semg, sems):
    wid = lax.axis_index("s") * 2 + lax.axis_index("c")
    base = wid * EPB
    pltpu.sync_copy(nbr_hbm.at[pl.ds(base, EPB)], idx_v)

    def gcopy(b):
        return pltpu.make_async_copy(
            y_hbm.at[idx_v.at[pl.ds(b * CH, CH)]], gbuf.at[b % 2], semg.at[b % 2])

    def scopy(b):
        return pltpu.make_async_copy(
            gbuf.at[b % 2], out_hbm.at[pl.ds(base + b * CH, CH)], sems.at[b % 2])

    gcopy(0).start()
    for b in range(NCH):
        gcopy(b).wait()
        if b + 1 < NCH:
            if b >= 1:
                scopy(b - 1).wait()
            gcopy(b + 1).start()
        scopy(b).start()
    scopy(NCH - 2).wait()
    scopy(NCH - 1).wait()


def _gelu_exact(v):
    z = v * np.float32(0.7071067811865476)
    a = jnp.abs(z)
    t = np.float32(1.0) / (np.float32(1.0) + np.float32(0.3275911) * a)
    p = np.float32(1.061405429)
    p = p * t + np.float32(-1.453152027)
    p = p * t + np.float32(1.421413741)
    p = p * t + np.float32(-0.284496736)
    p = p * t + np.float32(0.254829592)
    erf_a = np.float32(1.0) - (p * t) * jnp.exp(-a * a)
    erf = jnp.sign(z) * erf_a
    return np.float32(0.5) * v * (np.float32(1.0) + erf)


def _dense_a_body(pos_ref, feat_ref, w_in_ref, b_in_ref,
                  w_msg1_ref, omega_ref, cosm_ref, validm_ref, dimm_ref,
                  x_ref, y_ref):
    pos = pos_ref[...]
    px = pos[:, 0:1]
    py = pos[:, 1:2]
    pz = pos[:, 2:3]
    dimm = dimm_ref[...]
    posd = jnp.where(dimm == 0, px, jnp.where(dimm == 1, py, pz))
    arg = posd * omega_ref[...]
    emb = jnp.where(cosm_ref[...] > 0, jnp.cos(arg), jnp.sin(arg)) * validm_ref[...]
    x = emb + lax.dot_general(feat_ref[...], w_in_ref[...],
                              (((1,), (1,)), ((), ())),
                              preferred_element_type=jnp.float32) + b_in_ref[...]
    w1s = w_msg1_ref[:, :H]
    y = lax.dot_general(x, w1s, (((1,), (1,)), ((), ())),
                        preferred_element_type=jnp.float32)
    x_ref[...] = x
    y_ref[...] = y


def _dense_c_body(x_ref, gath_ref, w_ref, w_msg1_ref, b_msg1_ref,
                  w_msg2_ref, b_msg2_ref, w_proj_ref, b_proj_ref, out_ref):
    xs = x_ref[...]
    w1d = w_msg1_ref[:, H:]
    dstc = lax.dot_general(xs, w1d, (((1,), (1,)), ((), ())),
                           preferred_element_type=jnp.float32) + b_msg1_ref[...]
    drep = jnp.reshape(jnp.broadcast_to(dstc[:, None, :], (SB, K, H)), (SB * K, H))
    g = _gelu_exact(gath_ref[...] + drep) * w_ref[...]
    hs = jnp.sum(jnp.reshape(g, (SB, K, H)), axis=1)
    agg = lax.dot_general(hs, w_msg2_ref[...], (((1,), (1,)), ((), ())),
                          preferred_element_type=jnp.float32) + b_msg2_ref[...]
    wpa = w_proj_ref[:, :H]
    wps = w_proj_ref[:, H:]
    out = (lax.dot_general(agg, wpa, (((1,), (1,)), ((), ())),
                           preferred_element_type=jnp.float32)
           + lax.dot_general(xs, wps, (((1,), (1,)), ((), ())),
                             preferred_element_type=jnp.float32)
           + b_proj_ref[...])
    out_ref[0] = out


def _full(shape):
    n = len(shape)
    return pl.BlockSpec(shape, lambda i: (0,) * n)


def kernel(input_pos, input_feat, supernode_idx, W_in, b_in, W_msg1, b_msg1,
           W_msg2, b_msg2, W_proj, b_proj):
    pos = input_pos
    b_in2 = b_in.reshape(1, H)
    pos_pad = jnp.concatenate(
        [pos, jnp.full((NPAD - N, 3), 100.0, jnp.float32)], axis=0)
    px_a = pos_pad[:, 0]
    py_a = pos_pad[:, 1]
    pz_a = pos_pad[:, 2]

    nbr, w = pl.kernel(
        _sel_body,
        out_type=[jax.ShapeDtypeStruct((S * K,), jnp.int32),
                  jax.ShapeDtypeStruct((S * K,), jnp.float32)],
        mesh=plsc.VectorSubcoreMesh(core_axis_name="c", subcore_axis_name="s"),
        compiler_params=pltpu.CompilerParams(needs_layout_passes=False),
        scratch_types=[
            pltpu.VMEM((NPAD,), jnp.float32),
            pltpu.VMEM((NPAD,), jnp.float32),
            pltpu.VMEM((NPAD,), jnp.float32),
            pltpu.VMEM((CAP,), jnp.float32),
            pltpu.VMEM((CAP,), jnp.int32),
            pltpu.VMEM((EPB,), jnp.int32),
            pltpu.VMEM((EPB,), jnp.float32),
        ],
    )(px_a, py_a, pz_a)

    x, y = pl.pallas_call(
        _dense_a_body,
        grid=(NBLK,),
        in_specs=[
            pl.BlockSpec((ROWB, 3), lambda i: (i, 0)),
            pl.BlockSpec((ROWB, F), lambda i: (i, 0)),
            _full((H, F)),
            _full((1, H)),
            _full((H, 2 * H)),
            _full((1, H)),
            _full((1, H)),
            _full((1, H)),
            _full((1, H)),
        ],
        out_specs=[
            pl.BlockSpec((ROWB, H), lambda i: (i, 0)),
            pl.BlockSpec((ROWB, H), lambda i: (i, 0)),
        ],
        out_shape=[
            jax.ShapeDtypeStruct((N, H), jnp.float32),
            jax.ShapeDtypeStruct((N, H), jnp.float32),
        ],
    )(pos, input_feat, W_in, b_in2, W_msg1,
      jnp.asarray(_OMEGA), jnp.asarray(_COSM), jnp.asarray(_VALIDM), jnp.asarray(_DIMM))

    gath = pl.kernel(
        _gather_body,
        out_type=jax.ShapeDtypeStruct((S * K, H), jnp.float32),
        mesh=plsc.VectorSubcoreMesh(core_axis_name="c", subcore_axis_name="s"),
        compiler_params=pltpu.CompilerParams(needs_layout_passes=False),
        scratch_types=[
            pltpu.VMEM((EPB,), jnp.int32),
            pltpu.VMEM((2, CH, H), jnp.float32),
            pltpu.SemaphoreType.DMA((2,)),
            pltpu.SemaphoreType.DMA((2,)),
        ],
    )(y, nbr)

    out = pl.pallas_call(
        _dense_c_body,
        grid=(NSB,),
        in_specs=[
            pl.BlockSpec((SB, H), lambda i: (i, 0)),
            pl.BlockSpec((SB * K, H), lambda i: (i, 0)),
            pl.BlockSpec((SB * K, 1), lambda i: (i, 0)),
            _full((H, 2 * H)),
            _full((1, H)),
            _full((H, H)),
            _full((1, H)),
            _full((H, 2 * H)),
            _full((1, H)),
        ],
        out_specs=pl.BlockSpec((1, SB, H), lambda i: (0, i, 0)),
        out_shape=jax.ShapeDtypeStruct((1, S, H), jnp.float32),
    )(x[:S], gath, w.reshape(S * K, 1), W_msg1, b_msg1.reshape(1, H),
      W_msg2, b_msg2.reshape(1, H), W_proj, b_proj.reshape(1, H))
    return out


# speedup vs baseline: 6.9129x; 1.2041x over previous
import functools
import numpy as np
import jax
import jax.numpy as jnp
from jax import lax
from jax.experimental import pallas as pl
from jax.experimental.pallas import tpu as pltpu
from jax.experimental.pallas import tpu_sc as plsc

H = 256
F = 128
N = 10000
S = 1024
K = 32
R2 = np.float32(0.1 ** 2)
ROWB = 2000
NBLK = N // ROWB
NPAD = 10240
CB = NPAD // NBLK
SB = 128
NSB = S // SB
NEG_INF = np.float32(-np.inf)


def _embed_consts():
    eff = 84
    nfreq = eff // 2
    f = np.arange(H)
    d = np.minimum(f // (2 * nfreq), 2)
    r = f % (2 * nfreq)
    is_cos = (r >= nfreq).astype(np.float32)
    kk = r % nfreq
    omega = (1.0 / (10000.0 ** (np.arange(0, eff, 2).astype(np.float32) / np.float32(eff)))).astype(np.float32)
    omega_map = omega[kk]
    valid = (f < 6 * nfreq).astype(np.float32)
    return (omega_map.reshape(1, H), is_cos.reshape(1, H), valid.reshape(1, H),
            d.astype(np.int32).reshape(1, H))


_OMEGA, _COSM, _VALIDM, _DIMM = _embed_consts()


NW = 32
RPW = S // NW
CAP = 256
L = 16
GRP = 8
NVS = NPAD // L
EPB = S * K // NW
CH = 128
NCH = EPB // CH


RING = 4


def _row_gather(y_hbm, nbr_v, gbuf, semg, r):
    slot = lax.rem(r, RING)
    return pltpu.make_async_copy(
        y_hbm.at[nbr_v.at[pl.ds(r * K, K)]], gbuf.at[slot], semg.at[slot])


def _row_store(gbuf, gath_hbm, sems, gbase, r):
    slot = lax.rem(r, RING)
    return pltpu.make_async_copy(
        gbuf.at[slot], gath_hbm.at[pl.ds(gbase + r * K, K)], sems.at[slot])


def _lane(v, j=0):
    return lax.squeeze(lax.slice(v, (j,), (j + 1,)), (0,))


def _splat(s, dtype=jnp.int32):
    return jnp.full((L,), s, dtype)


def _sel_body(px_hbm, py_hbm, pz_hbm, y_hbm, gath_hbm, w_hbm,
              px, py, pz, cand_d, cand_i, nbr_v, w_v, gbuf, semg, sems):
    wid = lax.axis_index("s") * 2 + lax.axis_index("c")
    pltpu.sync_copy(px_hbm, px)
    pltpu.sync_copy(py_hbm, py)
    pltpu.sync_copy(pz_hbm, pz)
    iota = lax.iota(jnp.int32, L)
    zero_i = jnp.zeros((L,), jnp.int32)
    rowbase = wid * RPW

    cap_sp = jnp.full((L,), CAP, jnp.int32)
    capm1_sp = jnp.full((L,), CAP - 1, jnp.int32)

    def row_body(r, carry_dummy):
        i = rowbase + r
        sx = _splat(px[pl.ds(i, L)][0], jnp.float32)
        sy = _splat(py[pl.ds(i, L)][0], jnp.float32)
        sz = _splat(pz[pl.ds(i, L)][0], jnp.float32)

        def group_body(g, cnt_sp):
            off0 = g * (GRP * L)
            d2s = []
            ms = []
            anym = None
            for u in range(GRP):
                off = off0 + u * L
                dx = px[pl.ds(off, L)] - sx
                dy = py[pl.ds(off, L)] - sy
                dz = pz[pl.ds(off, L)] - sz
                d2 = (dx * dx + dy * dy) + dz * dz
                m = d2 <= R2
                d2s.append(d2)
                ms.append(m)
                anym = m if anym is None else (anym | m)
            anyc = _lane(plsc.cumsum(anym.astype(jnp.int32)), L - 1)

            def slow(cs):
                for u in range(GRP):
                    incl = plsc.cumsum(ms[u].astype(jnp.int32))
                    cu = _lane(incl, L - 1)

                    def hit(cs2, incl=incl, u=u, cu=cu, off=off0 + u * L):
                        tgt = jnp.minimum(cs2 + incl - 1, capm1_sp)
                        plsc.store_scatter(cand_d, [tgt], d2s[u], mask=ms[u])
                        plsc.store_scatter(cand_i, [tgt], iota + off, mask=ms[u])
                        return jnp.minimum(cs2 + _splat(cu), cap_sp)

                    cs = lax.cond(cu > 0, hit, lambda cs2: cs2, cs)
                return cs

            return lax.cond(anyc > 0, slow, lambda cs: cs, cnt_sp)

        cnt_sp = lax.fori_loop(0, NVS // GRP, group_body, zero_i)
        cnt = _lane(cnt_sp)
        nvc = (cnt + L - 1) // L

        def count_lt(thr_sp):
            def cbody(kk, acc):
                uv = plsc.bitcast(cand_d[pl.ds(kk * L, L)], jnp.int32)
                valid = (iota + kk * L) < cnt_sp
                mlt = (uv < thr_sp) & valid
                return acc + plsc.cumsum(mlt.astype(jnp.int32))
            acc = lax.fori_loop(0, nvc, cbody, zero_i)
            return _splat(_lane(acc, L - 1))

        def bit_body(bi, vthr):
            tryv = vthr | _splat(jnp.int32(1) << (30 - bi))
            return jnp.where(count_lt(tryv) < K, tryv, vthr)

        vthr = lax.fori_loop(0, 31, bit_body, zero_i)
        c_lt = count_lt(vthr)
        kp_sp = _splat(K, jnp.int32) - c_lt

        nbr_v[pl.ds(r * K, L)] = zero_i
        nbr_v[pl.ds(r * K + L, L)] = zero_i

        rk_sp = _splat(r * K)

        def emit_body(kk, carry):
            outp_sp, eqc = carry
            uv = plsc.bitcast(cand_d[pl.ds(kk * L, L)], jnp.int32)
            valid = (iota + kk * L) < cnt_sp
            lt = (uv < vthr) & valid
            eq = (uv == vthr) & valid
            pe = plsc.cumsum(eq.astype(jnp.int32)) + eqc
            sel = lt | (eq & (pe <= kp_sp))
            selc = plsc.cumsum(sel.astype(jnp.int32))
            tgt = rk_sp + outp_sp + selc - 1
            plsc.store_scatter(nbr_v, [tgt], cand_i[pl.ds(kk * L, L)], mask=sel)
            outp_sp = outp_sp + _splat(_lane(selc, L - 1))
            eqc = _splat(_lane(pe, L - 1))
            return outp_sp, eqc

        lax.fori_loop(0, nvc, emit_body, (zero_i, zero_i))

        ksel = jnp.minimum(cnt, K)
        inv_sp = jnp.full((L,), 1.0, jnp.float32) / _splat(ksel.astype(jnp.float32), jnp.float32)
        ksp = _splat(ksel)
        zf = jnp.zeros((L,), jnp.float32)
        w_v[pl.ds(r * K, L)] = jnp.where(iota < ksp, inv_sp, zf)
        w_v[pl.ds(r * K + L, L)] = jnp.where(iota + L < ksp, inv_sp, zf)

        slot = lax.rem(r, RING)

        @pl.when(r >= RING)
        def _():
            _row_store(gbuf, gath_hbm, sems, gbase, r - RING).wait()

        _row_gather(y_hbm, nbr_v, gbuf, semg, r).start()

        @pl.when(r >= 1)
        def _():
            _row_gather(y_hbm, nbr_v, gbuf, semg, r - 1).wait()
            _row_store(gbuf, gath_hbm, sems, gbase, r - 1).start()

        return 0

    gbase = wid * EPB
    lax.fori_loop(0, RPW, row_body, 0)
    _row_gather(y_hbm, nbr_v, gbuf, semg, RPW - 1).wait()
    _row_store(gbuf, gath_hbm, sems, gbase, RPW - 1).start()
    for d in range(RING, 0, -1):
        _row_store(gbuf, gath_hbm, sems, gbase, RPW - d).wait()
    pltpu.sync_copy(w_v, w_hbm.at[pl.ds(wid * EPB, EPB)])


def _gather_body(y_hbm, nbr_hbm, out_hbm, idx_v, gbuf, semg, sems):
    wid = lax.axis_index("s") * 2 + lax.axis_index("c")
    base = wid * EPB
    pltpu.sync_copy(nbr_hbm.at[pl.ds(base, EPB)], idx_v)

    def gcopy(b):
        return pltpu.make_async_copy(
            y_hbm.at[idx_v.at[pl.ds(b * CH, CH)]], gbuf.at[b % 2], semg.at[b % 2])

    def scopy(b):
        return pltpu.make_async_copy(
            gbuf.at[b % 2], out_hbm.at[pl.ds(base + b * CH, CH)], sems.at[b % 2])

    gcopy(0).start()
    for b in range(NCH):
        gcopy(b).wait()
        if b + 1 < NCH:
            if b >= 1:
                scopy(b - 1).wait()
            gcopy(b + 1).start()
        scopy(b).start()
    scopy(NCH - 2).wait()
    scopy(NCH - 1).wait()


def _gelu_exact(v):
    z = v * np.float32(0.7071067811865476)
    a = jnp.abs(z)
    t = np.float32(1.0) / (np.float32(1.0) + np.float32(0.3275911) * a)
    p = np.float32(1.061405429)
    p = p * t + np.float32(-1.453152027)
    p = p * t + np.float32(1.421413741)
    p = p * t + np.float32(-0.284496736)
    p = p * t + np.float32(0.254829592)
    erf_a = np.float32(1.0) - (p * t) * jnp.exp(-a * a)
    erf = jnp.sign(z) * erf_a
    return np.float32(0.5) * v * (np.float32(1.0) + erf)


def _dense_a_body(pos_ref, feat_ref, w_in_ref, b_in_ref,
                  w_msg1_ref, omega_ref, cosm_ref, validm_ref, dimm_ref,
                  x_ref, y_ref):
    pos = pos_ref[...]
    px = pos[:, 0:1]
    py = pos[:, 1:2]
    pz = pos[:, 2:3]
    dimm = dimm_ref[...]
    posd = jnp.where(dimm == 0, px, jnp.where(dimm == 1, py, pz))
    arg = posd * omega_ref[...]
    emb = jnp.where(cosm_ref[...] > 0, jnp.cos(arg), jnp.sin(arg)) * validm_ref[...]
    x = emb + lax.dot_general(feat_ref[...], w_in_ref[...],
                              (((1,), (1,)), ((), ())),
                              preferred_element_type=jnp.float32) + b_in_ref[...]
    w1s = w_msg1_ref[:, :H]
    y = lax.dot_general(x, w1s, (((1,), (1,)), ((), ())),
                        preferred_element_type=jnp.float32)
    x_ref[...] = x
    y_ref[...] = y


def _dense_c_body(x_ref, gath_ref, w_ref, w_msg1_ref, b_msg1_ref,
                  w_msg2_ref, b_msg2_ref, w_proj_ref, b_proj_ref, out_ref):
    xs = x_ref[...]
    w1d = w_msg1_ref[:, H:]
    dstc = lax.dot_general(xs, w1d, (((1,), (1,)), ((), ())),
                           preferred_element_type=jnp.float32) + b_msg1_ref[...]
    drep = jnp.reshape(jnp.broadcast_to(dstc[:, None, :], (SB, K, H)), (SB * K, H))
    g = _gelu_exact(gath_ref[...] + drep) * w_ref[...]
    hs = jnp.sum(jnp.reshape(g, (SB, K, H)), axis=1)
    agg = lax.dot_general(hs, w_msg2_ref[...], (((1,), (1,)), ((), ())),
                          preferred_element_type=jnp.float32) + b_msg2_ref[...]
    wpa = w_proj_ref[:, :H]
    wps = w_proj_ref[:, H:]
    out = (lax.dot_general(agg, wpa, (((1,), (1,)), ((), ())),
                           preferred_element_type=jnp.float32)
           + lax.dot_general(xs, wps, (((1,), (1,)), ((), ())),
                             preferred_element_type=jnp.float32)
           + b_proj_ref[...])
    out_ref[0] = out


def _full(shape):
    n = len(shape)
    return pl.BlockSpec(shape, lambda i: (0,) * n)


def kernel(input_pos, input_feat, supernode_idx, W_in, b_in, W_msg1, b_msg1,
           W_msg2, b_msg2, W_proj, b_proj):
    pos = input_pos
    b_in2 = b_in.reshape(1, H)
    pos_pad = jnp.concatenate(
        [pos, jnp.full((NPAD - N, 3), 100.0, jnp.float32)], axis=0)
    px_a = pos_pad[:, 0]
    py_a = pos_pad[:, 1]
    pz_a = pos_pad[:, 2]

    x, y = pl.pallas_call(
        _dense_a_body,
        grid=(NBLK,),
        in_specs=[
            pl.BlockSpec((ROWB, 3), lambda i: (i, 0)),
            pl.BlockSpec((ROWB, F), lambda i: (i, 0)),
            _full((H, F)),
            _full((1, H)),
            _full((H, 2 * H)),
            _full((1, H)),
            _full((1, H)),
            _full((1, H)),
            _full((1, H)),
        ],
        out_specs=[
            pl.BlockSpec((ROWB, H), lambda i: (i, 0)),
            pl.BlockSpec((ROWB, H), lambda i: (i, 0)),
        ],
        out_shape=[
            jax.ShapeDtypeStruct((N, H), jnp.float32),
            jax.ShapeDtypeStruct((N, H), jnp.float32),
        ],
    )(pos, input_feat, W_in, b_in2, W_msg1,
      jnp.asarray(_OMEGA), jnp.asarray(_COSM), jnp.asarray(_VALIDM), jnp.asarray(_DIMM))

    gath, w = pl.kernel(
        _sel_body,
        out_type=[jax.ShapeDtypeStruct((S * K, H), jnp.float32),
                  jax.ShapeDtypeStruct((S * K,), jnp.float32)],
        mesh=plsc.VectorSubcoreMesh(core_axis_name="c", subcore_axis_name="s"),
        compiler_params=pltpu.CompilerParams(needs_layout_passes=False),
        scratch_types=[
            pltpu.VMEM((NPAD,), jnp.float32),
            pltpu.VMEM((NPAD,), jnp.float32),
            pltpu.VMEM((NPAD,), jnp.float32),
            pltpu.VMEM((CAP,), jnp.float32),
            pltpu.VMEM((CAP,), jnp.int32),
            pltpu.VMEM((EPB,), jnp.int32),
            pltpu.VMEM((EPB,), jnp.float32),
            pltpu.VMEM((RING, K, H), jnp.float32),
            pltpu.SemaphoreType.DMA((RING,)),
            pltpu.SemaphoreType.DMA((RING,)),
        ],
    )(px_a, py_a, pz_a, y)

    out = pl.pallas_call(
        _dense_c_body,
        grid=(NSB,),
        in_specs=[
            pl.BlockSpec((SB, H), lambda i: (i, 0)),
            pl.BlockSpec((SB * K, H), lambda i: (i, 0)),
            pl.BlockSpec((SB * K, 1), lambda i: (i, 0)),
            _full((H, 2 * H)),
            _full((1, H)),
            _full((H, H)),
            _full((1, H)),
            _full((H, 2 * H)),
            _full((1, H)),
        ],
        out_specs=pl.BlockSpec((1, SB, H), lambda i: (0, i, 0)),
        out_shape=jax.ShapeDtypeStruct((1, S, H), jnp.float32),
    )(x[:S], gath, w.reshape(S * K, 1), W_msg1, b_msg1.reshape(1, H),
      W_msg2, b_msg2.reshape(1, H), W_proj, b_proj.reshape(1, H))
    return out
